# Initial kernel scaffold; baseline (speedup 1.0000x reference)
import jax
import jax.numpy as jnp
from jax.experimental import pallas as pl


def kernel(x0, x1, edge_index0, edge_index1, inter_edge_index, W_self0, W_neigh0, W_self1, W_neigh1, W_self_i, W_neigh_i):
    raise NotImplementedError("write your pallas kernel here")


# speedup vs baseline: 5.5811x; 5.5811x over previous
import jax
import jax.numpy as jnp
from jax import lax
from jax.experimental import pallas as pl
from jax.experimental.pallas import tpu as pltpu
from jax.experimental.pallas import tpu_sc as plsc

_N0, _E0 = 10000, 320000
_N1, _E1 = 2500, 40000
_D = 128
_NC, _NS = 2, 16
_NW = _NC * _NS
_CH = 80
_N0P = 10240
_N1P = 2560
_E1P = 40960
_DW = 16
_BM = 256

_mesh = plsc.VectorSubcoreMesh(core_axis_name="c", subcore_axis_name="s",
                               num_cores=_NC, num_subcores=_NS)
_sc_params = pltpu.CompilerParams(use_tc_tiling_on_sc=False)


def _seg_body(nch_w, rpt, nacc):
    def body(tab_hbm, src_hbm, dst_hbm, ones_hbm, z128_hbm, z16_hbm,
             agg_out, deg_out, sidx, didx, msg, ones_v, acc_sh, deg_sh, sem):
        cid = lax.axis_index("c")
        sid = lax.axis_index("s")
        wid = cid * _NS + sid
        pltpu.sync_copy(z128_hbm.at[pl.ds(0, rpt)],
                        acc_sh.at[pl.ds(sid * rpt, rpt)])
        pltpu.sync_copy(z16_hbm.at[pl.ds(0, rpt)],
                        deg_sh.at[pl.ds(sid * rpt, rpt)])
        pltpu.sync_copy(ones_hbm, ones_v)
        pltpu.sync_copy(src_hbm.at[pl.ds(wid * nch_w, nch_w)], sidx)
        pltpu.sync_copy(dst_hbm.at[pl.ds(wid * nch_w, nch_w)], didx)
        plsc.subcore_barrier()

        def step(c, carry):
            pltpu.async_copy(tab_hbm.at[sidx.at[c]], msg, sem).wait()
            pltpu.sync_copy(msg, acc_sh.at[didx.at[c]], add=True)
            pltpu.sync_copy(ones_v, deg_sh.at[didx.at[c]], add=True)
            return carry

        lax.fori_loop(0, nch_w, step, 0)
        plsc.subcore_barrier()
        pltpu.sync_copy(acc_sh.at[pl.ds(sid * rpt, rpt)],
                        agg_out.at[cid].at[pl.ds(sid * rpt, rpt)])
        pltpu.sync_copy(deg_sh.at[pl.ds(sid * rpt, rpt)],
                        deg_out.at[cid].at[pl.ds(sid * rpt, rpt)])
    return body


def _count_body(nch_w, rpt):
    def body(idx_hbm, ones_hbm, z16_hbm, cnt_out, cidx, ones_v, cnt_sh, sem):
        cid = lax.axis_index("c")
        sid = lax.axis_index("s")
        wid = cid * _NS + sid
        pltpu.sync_copy(z16_hbm.at[pl.ds(0, rpt)],
                        cnt_sh.at[pl.ds(sid * rpt, rpt)])
        pltpu.sync_copy(ones_hbm, ones_v)
        pltpu.sync_copy(idx_hbm.at[pl.ds(wid * nch_w, nch_w)], cidx)
        plsc.subcore_barrier()

        def step(c, carry):
            pltpu.sync_copy(ones_v, cnt_sh.at[cidx.at[c]], add=True)
            return carry

        lax.fori_loop(0, nch_w, step, 0)
        plsc.subcore_barrier()
        pltpu.sync_copy(cnt_sh.at[pl.ds(sid * rpt, rpt)],
                        cnt_out.at[cid].at[pl.ds(sid * rpt, rpt)])
    return body


def _inter_body(nch_w, rpt):
    def body(h1_hbm, h0_hbm, cl_hbm, z128_hbm, g1_out, csum_out,
             cidx, buf, buf2, csum_sh, sem):
        cid = lax.axis_index("c")
        sid = lax.axis_index("s")
        wid = cid * _NS + sid
        pltpu.sync_copy(z128_hbm.at[pl.ds(0, rpt)],
                        csum_sh.at[pl.ds(sid * rpt, rpt)])
        pltpu.sync_copy(cl_hbm.at[pl.ds(wid * nch_w, nch_w)], cidx)
        plsc.subcore_barrier()

        def step(c, carry):
            i0 = (wid * nch_w + c) * _CH
            pltpu.async_copy(h1_hbm.at[cidx.at[c]], buf, sem).wait()
            pltpu.sync_copy(buf, g1_out.at[pl.ds(i0, _CH)])
            pltpu.sync_copy(h0_hbm.at[pl.ds(i0, _CH)], buf2)
            pltpu.sync_copy(buf2, csum_sh.at[cidx.at[c]], add=True)
            return carry

        lax.fori_loop(0, nch_w, step, 0)
        plsc.subcore_barrier()
        pltpu.sync_copy(csum_sh.at[pl.ds(sid * rpt, rpt)],
                        csum_out.at[cid].at[pl.ds(sid * rpt, rpt)])
    return body


def _h_body(x_ref, p_ref, d_ref, ws_ref, wn_ref, o_ref):
    deg = d_ref[0] + d_ref[1]
    recip = 1.0 / jnp.maximum(deg[:, :1], 1.0)
    agg = (p_ref[0] + p_ref[1]) * recip
    o_ref[...] = jnp.maximum(
        jnp.dot(x_ref[...], ws_ref[...], preferred_element_type=jnp.float32)
        + jnp.dot(agg, wn_ref[...], preferred_element_type=jnp.float32), 0.0)


def _comb0_body(x_ref, h_ref, g_ref, wsi_ref, wni_ref, o_ref):
    h = h_ref[...]
    nz = jnp.maximum(
        jnp.dot(h, wsi_ref[...], preferred_element_type=jnp.float32)
        + jnp.dot(g_ref[...], wni_ref[...], preferred_element_type=jnp.float32),
        0.0)
    o_ref[...] = x_ref[...] + 0.5 * h + 0.5 * nz


def _comb1_body(x_ref, h_ref, c_ref, dc_ref, wsi_ref, wni_ref, o_ref):
    dc = dc_ref[0] + dc_ref[1]
    aggc = (c_ref[0] + c_ref[1]) * (1.0 / jnp.maximum(dc[:, :1], 1.0))
    h = h_ref[...]
    nz = jnp.maximum(
        jnp.dot(h, wsi_ref[...], preferred_element_type=jnp.float32)
        + jnp.dot(aggc, wni_ref[...], preferred_element_type=jnp.float32),
        0.0)
    o_ref[...] = x_ref[...] + 0.5 * h + 0.5 * nz


def _row_spec(bm, d):
    return pl.BlockSpec((bm, d), lambda i: (i, 0))


def _part_spec(bm, d):
    return pl.BlockSpec((2, bm, d), lambda i: (0, i, 0))


def _w_spec():
    return pl.BlockSpec((_D, _D), lambda i: (0, 0))


def _tc_h(xp, parts, degs, ws, wn, n_rows):
    return pl.pallas_call(
        _h_body,
        grid=(n_rows // _BM,),
        in_specs=[_row_spec(_BM, _D), _part_spec(_BM, _D),
                  _part_spec(_BM, _DW), _w_spec(), _w_spec()],
        out_specs=_row_spec(_BM, _D),
        out_shape=jax.ShapeDtypeStruct((n_rows, _D), jnp.float32),
    )(xp, parts, degs, ws, wn)


def kernel(x0, x1, edge_index0, edge_index1, inter_edge_index,
           W_self0, W_neigh0, W_self1, W_neigh1, W_self_i, W_neigh_i):
    f32 = jnp.float32
    x0p = jnp.pad(x0, ((0, _N0P - _N0), (0, 0)))
    x1p = jnp.pad(x1, ((0, _N1P - _N1), (0, 0)))
    src0 = edge_index0[0].reshape(_E0 // _CH, _CH)
    dst0 = edge_index0[1].reshape(_E0 // _CH, _CH)
    src1 = jnp.pad(edge_index1[0], (0, _E1P - _E1)).reshape(_E1P // _CH, _CH)
    dst1 = jnp.pad(edge_index1[1], (0, _E1P - _E1),
                   constant_values=_N1).reshape(_E1P // _CH, _CH)
    cluster = inter_edge_index[1, :_N0] - _N0
    clp = jnp.pad(cluster, (0, _N0P - _N0),
                  constant_values=_N1).reshape(_N0P // _CH, _CH)
    ones16 = jnp.ones((_CH, _DW), f32)
    z128 = jnp.zeros((_N0P // _NS, _D), f32)
    z16 = jnp.zeros((_N0P // _NS, _DW), f32)

    nch1 = _E1P // _CH // _NW
    rpt1 = _N1P // _NS
    agg1, deg1 = pl.kernel(
        _seg_body(nch1, rpt1, _N1P),
        out_type=(jax.ShapeDtypeStruct((_NC, _N1P, _D), f32),
                  jax.ShapeDtypeStruct((_NC, _N1P, _DW), f32)),
        mesh=_mesh,
        scratch_types=[
            pltpu.VMEM((nch1, _CH), jnp.int32),
            pltpu.VMEM((nch1, _CH), jnp.int32),
            pltpu.VMEM((_CH, _D), f32),
            pltpu.VMEM((_CH, _DW), f32),
            pltpu.VMEM_SHARED((_N1P, _D), f32),
            pltpu.VMEM_SHARED((_N1P, _DW), f32),
            pltpu.SemaphoreType.DMA,
        ],
        name="sc_seg_g1",
        compiler_params=_sc_params,
    )(x1, src1, dst1, ones16, z128, z16)

    nchc = _N0P // _CH // _NW
    degc = pl.kernel(
        _count_body(nchc, rpt1),
        out_type=jax.ShapeDtypeStruct((_NC, _N1P, _DW), f32),
        mesh=_mesh,
        scratch_types=[
            pltpu.VMEM((nchc, _CH), jnp.int32),
            pltpu.VMEM((_CH, _DW), f32),
            pltpu.VMEM_SHARED((_N1P, _DW), f32),
            pltpu.SemaphoreType.DMA,
        ],
        name="sc_count_cluster",
        compiler_params=_sc_params,
    )(clp, ones16, z16)

    nch0 = _E0 // _CH // _NW
    rpt0 = _N0P // _NS
    agg0, deg0 = pl.kernel(
        _seg_body(nch0, rpt0, _N0P),
        out_type=(jax.ShapeDtypeStruct((_NC, _N0P, _D), f32),
                  jax.ShapeDtypeStruct((_NC, _N0P, _DW), f32)),
        mesh=_mesh,
        scratch_types=[
            pltpu.VMEM((nch0, _CH), jnp.int32),
            pltpu.VMEM((nch0, _CH), jnp.int32),
            pltpu.VMEM((_CH, _D), f32),
            pltpu.VMEM((_CH, _DW), f32),
            pltpu.VMEM_SHARED((_N0P, _D), f32),
            pltpu.VMEM_SHARED((_N0P, _DW), f32),
            pltpu.SemaphoreType.DMA,
        ],
        name="sc_seg_g0",
        compiler_params=_sc_params,
    )(x0, src0, dst0, ones16, z128, z16)

    h1p = _tc_h(x1p, agg1, deg1, W_self1, W_neigh1, _N1P)
    h0p = _tc_h(x0p, agg0, deg0, W_self0, W_neigh0, _N0P)

    g1, csum = pl.kernel(
        _inter_body(nchc, rpt1),
        out_type=(jax.ShapeDtypeStruct((_N0P, _D), f32),
                  jax.ShapeDtypeStruct((_NC, _N1P, _D), f32)),
        mesh=_mesh,
        scratch_types=[
            pltpu.VMEM((nchc, _CH), jnp.int32),
            pltpu.VMEM((_CH, _D), f32),
            pltpu.VMEM((_CH, _D), f32),
            pltpu.VMEM_SHARED((_N1P, _D), f32),
            pltpu.SemaphoreType.DMA,
        ],
        name="sc_inter",
        compiler_params=_sc_params,
    )(h1p, h0p, clp, z128)

    out0p = pl.pallas_call(
        _comb0_body,
        grid=(_N0P // _BM,),
        in_specs=[_row_spec(_BM, _D), _row_spec(_BM, _D), _row_spec(_BM, _D),
                  _w_spec(), _w_spec()],
        out_specs=_row_spec(_BM, _D),
        out_shape=jax.ShapeDtypeStruct((_N0P, _D), f32),
    )(x0p, h0p, g1, W_self_i, W_neigh_i)

    out1p = pl.pallas_call(
        _comb1_body,
        grid=(_N1P // _BM,),
        in_specs=[_row_spec(_BM, _D), _row_spec(_BM, _D), _part_spec(_BM, _D),
                  _part_spec(_BM, _DW), _w_spec(), _w_spec()],
        out_specs=_row_spec(_BM, _D),
        out_shape=jax.ShapeDtypeStruct((_N1P, _D), f32),
    )(x1p, h1p, csum, degc, W_self_i, W_neigh_i)

    return (out0p[:_N0], out1p[:_N1])
